# Initial kernel scaffold; baseline (speedup 1.0000x reference)
import jax
import jax.numpy as jnp
from jax.experimental import pallas as pl


def kernel(z, supports, y_idx):
    raise NotImplementedError("write your pallas kernel here")


# speedup vs baseline: 4.4488x; 4.4488x over previous
import functools

import jax
import jax.numpy as jnp
from jax import lax
from jax.experimental import pallas as pl
from jax.experimental.pallas import tpu as pltpu
from jax.experimental.pallas import tpu_sc as plsc

TAU = 10.0
NCLS = 6
NQ = 1024
NS = 8192
D = 1024
TS = 512
NT = NS // TS
CPAD = 8
BIGI = 2 ** 30
PREC = lax.Precision.DEFAULT


def _main_body(z_ref, sup_ref, y_ref, idx_ref, log_ref, cn_ref,
               xn_ref, xnb_ref, xx_ref, bv_ref, cls_ref, cnt_ref):
    i = pl.program_id(0)

    @pl.when(i == 0)
    def _init():
        z = z_ref[...]
        zz = jnp.sum(z * z, axis=1, keepdims=True)
        xn = z / jnp.maximum(jnp.sqrt(zz), 1e-12)
        xn_ref[...] = xn
        xnb_ref[...] = xn.astype(jnp.bfloat16)
        xx_ref[...] = jnp.sum(xn * xn, axis=1, keepdims=True)
        cls_ref[...] = jnp.zeros_like(cls_ref)
        cnt_ref[...] = jnp.zeros_like(cnt_ref)

    sup = sup_ref[...]
    ss = jnp.sum(sup * sup, axis=1, keepdims=True)
    yn = sup / jnp.maximum(jnp.sqrt(ss), 1e-12)
    yy = jnp.sum(yn * yn, axis=1, keepdims=True)
    yy_row = jnp.transpose(yy)
    xy = lax.dot_general(xnb_ref[...], yn.astype(jnp.bfloat16),
                         (((1,), (1,)), ((), ())),
                         preferred_element_type=jnp.float32)
    dist = (xx_ref[...] + yy_row) - 2.0 * xy

    m = jnp.min(dist, axis=1, keepdims=True)
    cols = lax.broadcasted_iota(jnp.int32, (NQ, TS), 1) + i * TS
    am = jnp.min(jnp.where(dist == m, cols, BIGI), axis=1, keepdims=True)

    @pl.when(i == 0)
    def _first():
        bv_ref[...] = m
        idx_ref[...] = am

    @pl.when(i > 0)
    def _merge():
        better = m < bv_ref[...]
        bv_ref[...] = jnp.where(better, m, bv_ref[...])
        idx_ref[...] = jnp.where(better, am, idx_ref[...])

    y = jnp.broadcast_to(y_ref[0], (CPAD, TS))
    oh = (lax.broadcasted_iota(jnp.int32, (CPAD, TS), 0) == y
          ).astype(jnp.float32)
    cls_ref[...] += lax.dot_general(oh, sup, (((1,), (0,)), ((), ())),
                                    precision=PREC)
    cnt_ref[...] += jnp.sum(oh, axis=1, keepdims=True)

    @pl.when(i == NT - 1)
    def _tail():
        cnt = cnt_ref[...]
        recip = (1.0 / (cnt + 1e-12)).astype(jnp.bfloat16).astype(jnp.float32)
        cen = cls_ref[...] * recip
        cc = jnp.sum(cen * cen, axis=1, keepdims=True)
        cn = cen / jnp.maximum(jnp.sqrt(cc), 1e-12)
        cn_ref[...] = cn
        log_ref[...] = lax.dot_general(
            TAU * xn_ref[...], cn, (((1,), (1,)), ((), ())),
            precision=PREC)


_main_call = pl.pallas_call(
    _main_body,
    grid=(NT,),
    in_specs=[
        pl.BlockSpec((NQ, D), lambda i: (0, 0)),
        pl.BlockSpec((TS, D), lambda i: (i, 0)),
        pl.BlockSpec((1, 1, TS), lambda i: (i, 0, 0)),
    ],
    out_specs=[
        pl.BlockSpec((NQ, 1), lambda i: (0, 0)),
        pl.BlockSpec((NQ, CPAD), lambda i: (0, 0)),
        pl.BlockSpec((CPAD, D), lambda i: (0, 0)),
    ],
    out_shape=[
        jax.ShapeDtypeStruct((NQ, 1), jnp.int32),
        jax.ShapeDtypeStruct((NQ, CPAD), jnp.float32),
        jax.ShapeDtypeStruct((CPAD, D), jnp.float32),
    ],
    scratch_shapes=[
        pltpu.VMEM((NQ, D), jnp.float32),
        pltpu.VMEM((NQ, D), jnp.bfloat16),
        pltpu.VMEM((NQ, 1), jnp.float32),
        pltpu.VMEM((NQ, 1), jnp.float32),
        pltpu.VMEM((CPAD, D), jnp.float32),
        pltpu.VMEM((CPAD, 1), jnp.float32),
    ],
)


def _finish_body(sel_ref, cn_ref, tgt_ref, out_ref):
    cn = cn_ref[...]
    sel = sel_ref[...]
    sz = jnp.sum(sel * sel, axis=1, keepdims=True)
    sn = sel / jnp.maximum(jnp.sqrt(sz), 1e-12)
    tl = lax.dot_general(
        TAU * sn, cn, (((1,), (1,)), ((), ())), precision=PREC)

    col = lax.broadcasted_iota(jnp.int32, (NQ, CPAD), 1)
    valid = col < NCLS
    tlm = jnp.where(valid, tl, jnp.float32(-3.0e38))
    mx = jnp.max(tlm, axis=1, keepdims=True)
    am = jnp.min(jnp.where(tlm == mx, col, BIGI), axis=1, keepdims=True)
    onehot = (col == am).astype(jnp.float32)
    tgt_ref[...] = onehot / (jnp.sum(onehot, axis=1, keepdims=True) + 1e-12)

    e = jnp.where(valid, jnp.exp(tl - mx), 0.0)
    sm = e / jnp.sum(e, axis=1, keepdims=True)
    out_ref[...] = sm / (jnp.sum(sm, axis=1, keepdims=True) + 1e-12)


_finish_call = pl.pallas_call(
    _finish_body,
    out_shape=[
        jax.ShapeDtypeStruct((NQ, CPAD), jnp.float32),
        jax.ShapeDtypeStruct((NQ, CPAD), jnp.float32),
    ],
)


_SC_NC = 2
_SC_NS = 16
_NW = _SC_NC * _SC_NS
_BPW = NQ // _NW


def _sc_gather_body(table_hbm, idx_hbm, out_hbm, idx_v, rows_v, sem):
    wid = lax.axis_index("s") * _SC_NC + lax.axis_index("c")
    base = wid * _BPW
    pltpu.sync_copy(idx_hbm.at[pl.ds(base, _BPW)], idx_v)
    pltpu.async_copy(table_hbm.at[idx_v], rows_v, sem).wait()
    pltpu.sync_copy(rows_v, out_hbm.at[pl.ds(base, _BPW)])


@functools.cache
def _sc_gather_call():
    return pl.kernel(
        _sc_gather_body,
        mesh=plsc.VectorSubcoreMesh(core_axis_name="c", subcore_axis_name="s",
                                    num_cores=_SC_NC, num_subcores=_SC_NS),
        out_type=jax.ShapeDtypeStruct((NQ, D), jnp.float32),
        scratch_types=[
            pltpu.VMEM((_BPW,), jnp.int32),
            pltpu.VMEM((_BPW, D), jnp.float32),
            pltpu.SemaphoreType.DMA,
        ],
    )


def kernel(z, supports, y_idx):
    y3 = y_idx.astype(jnp.int32).reshape(NT, 1, TS)
    idx, lg, cn = _main_call(z, supports, y3)
    sel = _sc_gather_call()(supports, idx.reshape(NQ))
    tg, ou = _finish_call(sel, cn)
    return (lg[:, :NCLS], tg[:, :NCLS], ou[:, :NCLS])


# speedup vs baseline: 4.7434x; 1.0662x over previous
import functools

import jax
import jax.numpy as jnp
from jax import lax
from jax.experimental import pallas as pl
from jax.experimental.pallas import tpu as pltpu
from jax.experimental.pallas import tpu_sc as plsc

TAU = 10.0
NCLS = 6
NQ = 1024
NS = 8192
D = 1024
TS = 1024
NT = NS // TS
CPAD = 8
BIGI = 2 ** 30
PREC = lax.Precision.DEFAULT


def _main_body(z_ref, sup_ref, y_ref, idx_ref, log_ref, cn_ref,
               xn_ref, xnb_ref, xx_ref, bv_ref, cls_ref, cnt_ref):
    i = pl.program_id(0)

    @pl.when(i == 0)
    def _init():
        z = z_ref[...]
        zz = jnp.sum(z * z, axis=1, keepdims=True)
        xn = z / jnp.maximum(jnp.sqrt(zz), 1e-12)
        xn_ref[...] = xn
        xnb_ref[...] = xn.astype(jnp.bfloat16)
        xx_ref[...] = jnp.sum(xn * xn, axis=1, keepdims=True)
        cls_ref[...] = jnp.zeros_like(cls_ref)
        cnt_ref[...] = jnp.zeros_like(cnt_ref)

    sup = sup_ref[...]
    ss = jnp.sum(sup * sup, axis=1, keepdims=True)
    yn = sup / jnp.maximum(jnp.sqrt(ss), 1e-12)
    yy = jnp.sum(yn * yn, axis=1, keepdims=True)
    yy_row = jnp.transpose(yy)
    xy = lax.dot_general(xnb_ref[...], yn.astype(jnp.bfloat16),
                         (((1,), (1,)), ((), ())),
                         preferred_element_type=jnp.float32)
    dist = (xx_ref[...] + yy_row) - 2.0 * xy

    m = jnp.min(dist, axis=1, keepdims=True)
    cols = lax.broadcasted_iota(jnp.int32, (NQ, TS), 1) + i * TS
    am = jnp.min(jnp.where(dist == m, cols, BIGI), axis=1, keepdims=True)

    @pl.when(i == 0)
    def _first():
        bv_ref[...] = m
        idx_ref[...] = am

    @pl.when(i > 0)
    def _merge():
        better = m < bv_ref[...]
        bv_ref[...] = jnp.where(better, m, bv_ref[...])
        idx_ref[...] = jnp.where(better, am, idx_ref[...])

    y = jnp.broadcast_to(y_ref[0], (CPAD, TS))
    oh = (lax.broadcasted_iota(jnp.int32, (CPAD, TS), 0) == y
          ).astype(jnp.float32)
    cls_ref[...] += lax.dot_general(oh, sup, (((1,), (0,)), ((), ())),
                                    precision=PREC)
    cnt_ref[...] += jnp.sum(oh, axis=1, keepdims=True)

    @pl.when(i == NT - 1)
    def _tail():
        cnt = cnt_ref[...]
        recip = (1.0 / (cnt + 1e-12)).astype(jnp.bfloat16).astype(jnp.float32)
        cen = cls_ref[...] * recip
        cc = jnp.sum(cen * cen, axis=1, keepdims=True)
        cn = cen / jnp.maximum(jnp.sqrt(cc), 1e-12)
        cn_ref[...] = cn
        log_ref[...] = lax.dot_general(
            TAU * xn_ref[...], cn, (((1,), (1,)), ((), ())),
            precision=PREC)


_main_call = pl.pallas_call(
    _main_body,
    grid=(NT,),
    in_specs=[
        pl.BlockSpec((NQ, D), lambda i: (0, 0)),
        pl.BlockSpec((TS, D), lambda i: (i, 0)),
        pl.BlockSpec((1, 1, TS), lambda i: (i, 0, 0)),
    ],
    out_specs=[
        pl.BlockSpec((NQ, 1), lambda i: (0, 0)),
        pl.BlockSpec((NQ, CPAD), lambda i: (0, 0)),
        pl.BlockSpec((CPAD, D), lambda i: (0, 0)),
    ],
    out_shape=[
        jax.ShapeDtypeStruct((NQ, 1), jnp.int32),
        jax.ShapeDtypeStruct((NQ, CPAD), jnp.float32),
        jax.ShapeDtypeStruct((CPAD, D), jnp.float32),
    ],
    scratch_shapes=[
        pltpu.VMEM((NQ, D), jnp.float32),
        pltpu.VMEM((NQ, D), jnp.bfloat16),
        pltpu.VMEM((NQ, 1), jnp.float32),
        pltpu.VMEM((NQ, 1), jnp.float32),
        pltpu.VMEM((CPAD, D), jnp.float32),
        pltpu.VMEM((CPAD, 1), jnp.float32),
    ],
)


def _finish_body(sel_ref, cn_ref, tgt_ref, out_ref):
    cn = cn_ref[...]
    sel = sel_ref[...]
    sz = jnp.sum(sel * sel, axis=1, keepdims=True)
    sn = sel / jnp.maximum(jnp.sqrt(sz), 1e-12)
    tl = lax.dot_general(
        TAU * sn, cn, (((1,), (1,)), ((), ())), precision=PREC)

    col = lax.broadcasted_iota(jnp.int32, (NQ, CPAD), 1)
    valid = col < NCLS
    tlm = jnp.where(valid, tl, jnp.float32(-3.0e38))
    mx = jnp.max(tlm, axis=1, keepdims=True)
    am = jnp.min(jnp.where(tlm == mx, col, BIGI), axis=1, keepdims=True)
    onehot = (col == am).astype(jnp.float32)
    tgt_ref[...] = onehot / (jnp.sum(onehot, axis=1, keepdims=True) + 1e-12)

    e = jnp.where(valid, jnp.exp(tl - mx), 0.0)
    sm = e / jnp.sum(e, axis=1, keepdims=True)
    out_ref[...] = sm / (jnp.sum(sm, axis=1, keepdims=True) + 1e-12)


_finish_call = pl.pallas_call(
    _finish_body,
    out_shape=[
        jax.ShapeDtypeStruct((NQ, CPAD), jnp.float32),
        jax.ShapeDtypeStruct((NQ, CPAD), jnp.float32),
    ],
)


_SC_NC = 2
_SC_NS = 16
_NW = _SC_NC * _SC_NS
_BPW = NQ // _NW


def _sc_gather_body(table_hbm, idx_hbm, out_hbm, idx_v, rows_v, sem):
    wid = lax.axis_index("s") * _SC_NC + lax.axis_index("c")
    base = wid * _BPW
    pltpu.sync_copy(idx_hbm.at[pl.ds(base, _BPW)], idx_v)
    pltpu.async_copy(table_hbm.at[idx_v], rows_v, sem).wait()
    pltpu.sync_copy(rows_v, out_hbm.at[pl.ds(base, _BPW)])


@functools.cache
def _sc_gather_call():
    return pl.kernel(
        _sc_gather_body,
        mesh=plsc.VectorSubcoreMesh(core_axis_name="c", subcore_axis_name="s",
                                    num_cores=_SC_NC, num_subcores=_SC_NS),
        out_type=jax.ShapeDtypeStruct((NQ, D), jnp.float32),
        scratch_types=[
            pltpu.VMEM((_BPW,), jnp.int32),
            pltpu.VMEM((_BPW, D), jnp.float32),
            pltpu.SemaphoreType.DMA,
        ],
    )


def kernel(z, supports, y_idx):
    y3 = y_idx.astype(jnp.int32).reshape(NT, 1, TS)
    idx, lg, cn = _main_call(z, supports, y3)
    sel = _sc_gather_call()(supports, idx.reshape(NQ))
    tg, ou = _finish_call(sel, cn)
    return (lg[:, :NCLS], tg[:, :NCLS], ou[:, :NCLS])


# speedup vs baseline: 4.7840x; 1.0086x over previous
import functools

import jax
import jax.numpy as jnp
from jax import lax
from jax.experimental import pallas as pl
from jax.experimental.pallas import tpu as pltpu
from jax.experimental.pallas import tpu_sc as plsc

TAU = 10.0
NCLS = 6
NQ = 1024
NS = 8192
D = 1024
TS = 2048
NT = NS // TS
CPAD = 8
BIGI = 2 ** 30
PREC = lax.Precision.DEFAULT


def _main_body(z_ref, sup_ref, y_ref, idx_ref, log_ref, cn_ref,
               xn_ref, xnb_ref, xx_ref, bv_ref, cls_ref, cnt_ref):
    i = pl.program_id(0)

    @pl.when(i == 0)
    def _init():
        z = z_ref[...]
        zz = jnp.sum(z * z, axis=1, keepdims=True)
        xn = z / jnp.maximum(jnp.sqrt(zz), 1e-12)
        xn_ref[...] = xn
        xnb_ref[...] = xn.astype(jnp.bfloat16)
        xx_ref[...] = jnp.sum(xn * xn, axis=1, keepdims=True)
        cls_ref[...] = jnp.zeros_like(cls_ref)
        cnt_ref[...] = jnp.zeros_like(cnt_ref)

    sup = sup_ref[...]
    ss = jnp.sum(sup * sup, axis=1, keepdims=True)
    yn = sup / jnp.maximum(jnp.sqrt(ss), 1e-12)
    yy = jnp.sum(yn * yn, axis=1, keepdims=True)
    yy_row = jnp.transpose(yy)
    xy = lax.dot_general(xnb_ref[...], yn.astype(jnp.bfloat16),
                         (((1,), (1,)), ((), ())),
                         preferred_element_type=jnp.float32)
    dist = (xx_ref[...] + yy_row) - 2.0 * xy

    m = jnp.min(dist, axis=1, keepdims=True)
    cols = lax.broadcasted_iota(jnp.int32, (NQ, TS), 1) + i * TS
    am = jnp.min(jnp.where(dist == m, cols, BIGI), axis=1, keepdims=True)

    @pl.when(i == 0)
    def _first():
        bv_ref[...] = m
        idx_ref[...] = am

    @pl.when(i > 0)
    def _merge():
        better = m < bv_ref[...]
        bv_ref[...] = jnp.where(better, m, bv_ref[...])
        idx_ref[...] = jnp.where(better, am, idx_ref[...])

    y = jnp.broadcast_to(y_ref[0], (CPAD, TS))
    oh = (lax.broadcasted_iota(jnp.int32, (CPAD, TS), 0) == y
          ).astype(jnp.float32)
    cls_ref[...] += lax.dot_general(oh, sup, (((1,), (0,)), ((), ())),
                                    precision=PREC)
    cnt_ref[...] += jnp.sum(oh, axis=1, keepdims=True)

    @pl.when(i == NT - 1)
    def _tail():
        cnt = cnt_ref[...]
        recip = (1.0 / (cnt + 1e-12)).astype(jnp.bfloat16).astype(jnp.float32)
        cen = cls_ref[...] * recip
        cc = jnp.sum(cen * cen, axis=1, keepdims=True)
        cn = cen / jnp.maximum(jnp.sqrt(cc), 1e-12)
        cn_ref[...] = cn
        log_ref[...] = lax.dot_general(
            TAU * xn_ref[...], cn, (((1,), (1,)), ((), ())),
            precision=PREC)


_main_call = pl.pallas_call(
    _main_body,
    grid=(NT,),
    in_specs=[
        pl.BlockSpec((NQ, D), lambda i: (0, 0)),
        pl.BlockSpec((TS, D), lambda i: (i, 0)),
        pl.BlockSpec((1, 1, TS), lambda i: (i, 0, 0)),
    ],
    out_specs=[
        pl.BlockSpec((NQ, 1), lambda i: (0, 0)),
        pl.BlockSpec((NQ, CPAD), lambda i: (0, 0)),
        pl.BlockSpec((CPAD, D), lambda i: (0, 0)),
    ],
    out_shape=[
        jax.ShapeDtypeStruct((NQ, 1), jnp.int32),
        jax.ShapeDtypeStruct((NQ, CPAD), jnp.float32),
        jax.ShapeDtypeStruct((CPAD, D), jnp.float32),
    ],
    scratch_shapes=[
        pltpu.VMEM((NQ, D), jnp.float32),
        pltpu.VMEM((NQ, D), jnp.bfloat16),
        pltpu.VMEM((NQ, 1), jnp.float32),
        pltpu.VMEM((NQ, 1), jnp.float32),
        pltpu.VMEM((CPAD, D), jnp.float32),
        pltpu.VMEM((CPAD, 1), jnp.float32),
    ],
)


def _finish_body(sel_ref, cn_ref, tgt_ref, out_ref):
    cn = cn_ref[...]
    sel = sel_ref[...]
    sz = jnp.sum(sel * sel, axis=1, keepdims=True)
    sn = sel / jnp.maximum(jnp.sqrt(sz), 1e-12)
    tl = lax.dot_general(
        TAU * sn, cn, (((1,), (1,)), ((), ())), precision=PREC)

    col = lax.broadcasted_iota(jnp.int32, (NQ, CPAD), 1)
    valid = col < NCLS
    tlm = jnp.where(valid, tl, jnp.float32(-3.0e38))
    mx = jnp.max(tlm, axis=1, keepdims=True)
    am = jnp.min(jnp.where(tlm == mx, col, BIGI), axis=1, keepdims=True)
    onehot = (col == am).astype(jnp.float32)
    tgt_ref[...] = onehot / (jnp.sum(onehot, axis=1, keepdims=True) + 1e-12)

    e = jnp.where(valid, jnp.exp(tl - mx), 0.0)
    sm = e / jnp.sum(e, axis=1, keepdims=True)
    out_ref[...] = sm / (jnp.sum(sm, axis=1, keepdims=True) + 1e-12)


_finish_call = pl.pallas_call(
    _finish_body,
    out_shape=[
        jax.ShapeDtypeStruct((NQ, CPAD), jnp.float32),
        jax.ShapeDtypeStruct((NQ, CPAD), jnp.float32),
    ],
)


_SC_NC = 2
_SC_NS = 16
_NW = _SC_NC * _SC_NS
_BPW = NQ // _NW


def _sc_gather_body(table_hbm, idx_hbm, out_hbm, idx_v, rows_v, sem):
    wid = lax.axis_index("s") * _SC_NC + lax.axis_index("c")
    base = wid * _BPW
    pltpu.sync_copy(idx_hbm.at[pl.ds(base, _BPW)], idx_v)
    pltpu.async_copy(table_hbm.at[idx_v], rows_v, sem).wait()
    pltpu.sync_copy(rows_v, out_hbm.at[pl.ds(base, _BPW)])


@functools.cache
def _sc_gather_call():
    return pl.kernel(
        _sc_gather_body,
        mesh=plsc.VectorSubcoreMesh(core_axis_name="c", subcore_axis_name="s",
                                    num_cores=_SC_NC, num_subcores=_SC_NS),
        out_type=jax.ShapeDtypeStruct((NQ, D), jnp.float32),
        scratch_types=[
            pltpu.VMEM((_BPW,), jnp.int32),
            pltpu.VMEM((_BPW, D), jnp.float32),
            pltpu.SemaphoreType.DMA,
        ],
    )


def kernel(z, supports, y_idx):
    y3 = y_idx.astype(jnp.int32).reshape(NT, 1, TS)
    idx, lg, cn = _main_call(z, supports, y3)
    sel = _sc_gather_call()(supports, idx.reshape(NQ))
    tg, ou = _finish_call(sel, cn)
    return (lg[:, :NCLS], tg[:, :NCLS], ou[:, :NCLS])
